# baseline (speedup 1.0000x reference)
import jax
import jax.numpy as jnp
from jax import lax
from jax.experimental import pallas as pl
from jax.experimental.pallas import tpu as pltpu
from jax.experimental.pallas import tpu_sc as plsc

_B = 16384
_D = 64
_NW = 32
_BPW = _B // _NW
_ICH = 128
_NCH = _BPW // _ICH
_GROUPS = _BPW // 16


def _body(uidx_hbm, iidx_hbm, utab_hbm, itab_hbm, w_hbm, b_hbm, out_hbm,
          uidx_v, iidx_v, ug_v, ig_v, rows_v, w_v, b_v, out_v, sem):
    wid = lax.axis_index("s") * 2 + lax.axis_index("c")
    base = wid * _BPW

    pltpu.sync_copy(uidx_hbm.at[pl.ds(base, _BPW)], uidx_v)
    pltpu.sync_copy(iidx_hbm.at[pl.ds(base, _BPW)], iidx_v)
    pltpu.sync_copy(w_hbm, w_v)
    pltpu.sync_copy(b_hbm, b_v)

    def halve(c, carry):
        ug_v[pl.ds(c * 16, 16)] = lax.shift_right_logical(
            uidx_v[pl.ds(c * 16, 16)], 1)
        ig_v[pl.ds(c * 16, 16)] = lax.shift_right_logical(
            iidx_v[pl.ds(c * 16, 16)], 1)
        return carry
    lax.fori_loop(0, _GROUPS, halve, 0)

    lane = lax.broadcasted_iota(jnp.int32, (16,), 0)
    bv = b_v[pl.ds(0, 16)]
    wv = [w_v[pl.ds(c * 16, 16)] for c in range(8)]

    def gather_rows(tab_hbm, g_v):
        copies = [
            pltpu.async_copy(tab_hbm.at[g_v.at[pl.ds(k * _ICH, _ICH)]],
                             rows_v.at[pl.ds(k * _ICH, _ICH)], sem)
            for k in range(_NCH)
        ]
        for c in copies:
            c.wait()

    def compute_phase(idx_v, wbank, first):
        def group(g, carry):
            idx_vals = idx_v[pl.ds(g * 16, 16)]
            par = idx_vals & 1
            acc = jnp.zeros((16,), jnp.float32)
            parf = par.astype(jnp.float32)
            for rr in range(16):
                r = g * 16 + rr
                m = parf.at[jnp.full((16,), rr, jnp.int32)].get(
                    mode="promise_in_bounds")
                p0 = rows_v[r, pl.ds(0, 16)] * wbank[0]
                p1 = rows_v[r, pl.ds(64, 16)] * wbank[0]
                for c in range(1, 4):
                    p0 = p0 + rows_v[r, pl.ds(c * 16, 16)] * wbank[c]
                    p1 = p1 + rows_v[r, pl.ds(64 + c * 16, 16)] * wbank[c]
                p = p0 + m * (p1 - p0)
                for d in (1, 2, 4, 8):
                    p = p + p.at[lane ^ d].get(mode="promise_in_bounds")
                acc = jnp.where(lane == rr, p, acc)
            if first:
                out_v[pl.ds(g * 16, 16)] = acc + bv
            else:
                out_v[pl.ds(g * 16, 16)] = out_v[pl.ds(g * 16, 16)] + acc
            return carry
        lax.fori_loop(0, _GROUPS, group, 0)

    gather_rows(utab_hbm, ug_v)
    compute_phase(uidx_v, wv[:4], True)
    gather_rows(itab_hbm, ig_v)
    compute_phase(iidx_v, wv[4:], False)

    pltpu.sync_copy(out_v, out_hbm.at[pl.ds(base, _BPW)])


def kernel(user_idx, item_idx, user_table, item_table, W, b):
    wf = W.reshape(-1).astype(jnp.float32)
    b16 = jnp.broadcast_to(b.astype(jnp.float32), (128,))
    ut2 = user_table.reshape(user_table.shape[0] // 2, 2 * _D)
    it2 = item_table.reshape(item_table.shape[0] // 2, 2 * _D)
    mesh = plsc.VectorSubcoreMesh(core_axis_name="c", subcore_axis_name="s")
    f = pl.kernel(
        _body,
        out_type=jax.ShapeDtypeStruct((_B,), jnp.float32),
        mesh=mesh,
        compiler_params=pltpu.CompilerParams(use_tc_tiling_on_sc=True),
        scratch_types=[
            pltpu.VMEM((_BPW,), jnp.int32),
            pltpu.VMEM((_BPW,), jnp.int32),
            pltpu.VMEM((_BPW,), jnp.int32),
            pltpu.VMEM((_BPW,), jnp.int32),
            pltpu.VMEM((_BPW, 2 * _D), jnp.float32),
            pltpu.VMEM((2 * _D,), jnp.float32),
            pltpu.VMEM((128,), jnp.float32),
            pltpu.VMEM((_BPW,), jnp.float32),
            pltpu.SemaphoreType.DMA,
        ],
    )
    return f(user_idx.astype(jnp.int32), item_idx.astype(jnp.int32),
             ut2, it2, wf, b16)


# speedup vs baseline: 1.6596x; 1.6596x over previous
import jax
import jax.numpy as jnp
from jax import lax
from jax.experimental import pallas as pl
from jax.experimental.pallas import tpu as pltpu
from jax.experimental.pallas import tpu_sc as plsc

_B = 16384
_D = 64
_NW = 32
_BPW = _B // _NW
_GROUPS = _BPW // 16


def _body(uidx_hbm, iidx_hbm, utab_hbm, itab_hbm, w_hbm, b_hbm, out_hbm,
          uidx_v, iidx_v, rows_v, w_v, b_v, out_v, sem):
    wid = lax.axis_index("s") * 2 + lax.axis_index("c")
    base = wid * _BPW

    pltpu.sync_copy(uidx_hbm.at[pl.ds(base, _BPW)], uidx_v)
    pltpu.sync_copy(iidx_hbm.at[pl.ds(base, _BPW)], iidx_v)
    pltpu.sync_copy(w_hbm, w_v)
    pltpu.sync_copy(b_hbm, b_v)

    lane = lax.broadcasted_iota(jnp.int32, (16,), 0)
    bv = b_v[pl.ds(0, 16)]
    wv = [w_v[pl.ds(c * 16, 16)] for c in range(8)]

    def phase(tab_hbm, idx_s, wbank, first):
        def fire(g, carry):
            vals = idx_s[pl.ds(g * 16, 16)]
            for rr in range(16):
                pltpu.async_copy(tab_hbm.at[pl.ds(vals[rr], 1)],
                                 rows_v.at[pl.ds(g * 16 + rr, 1)], sem)
            return carry
        lax.fori_loop(0, _GROUPS, fire, 0)

        def drain(g, carry):
            for rr in range(16):
                pltpu.make_async_copy(tab_hbm.at[pl.ds(0, 1)],
                                      rows_v.at[pl.ds(0, 1)], sem).wait()
            return carry
        lax.fori_loop(0, _GROUPS, drain, 0)

        def group(g, carry):
            acc = jnp.zeros((16,), jnp.float32)
            for rr in range(16):
                r = g * 16 + rr
                p = rows_v[r, pl.ds(0, 16)] * wbank[0]
                for c in range(1, 4):
                    p = p + rows_v[r, pl.ds(c * 16, 16)] * wbank[c]
                for d in (1, 2, 4, 8):
                    p = p + p.at[lane ^ d].get(mode="promise_in_bounds")
                acc = jnp.where(lane == rr, p, acc)
            if first:
                out_v[pl.ds(g * 16, 16)] = acc + bv
            else:
                out_v[pl.ds(g * 16, 16)] = out_v[pl.ds(g * 16, 16)] + acc
            return carry
        lax.fori_loop(0, _GROUPS, group, 0)

    phase(utab_hbm, uidx_v, wv[:4], True)
    phase(itab_hbm, iidx_v, wv[4:], False)

    pltpu.sync_copy(out_v, out_hbm.at[pl.ds(base, _BPW)])


def kernel(user_idx, item_idx, user_table, item_table, W, b):
    wf = W.reshape(-1).astype(jnp.float32)
    b128 = jnp.broadcast_to(b.astype(jnp.float32), (128,))
    mesh = plsc.VectorSubcoreMesh(core_axis_name="c", subcore_axis_name="s")
    f = pl.kernel(
        _body,
        out_type=jax.ShapeDtypeStruct((_B,), jnp.float32),
        mesh=mesh,
        compiler_params=pltpu.CompilerParams(use_tc_tiling_on_sc=True),
        scratch_types=[
            pltpu.VMEM((_BPW,), jnp.int32),
            pltpu.VMEM((_BPW,), jnp.int32),
            pltpu.VMEM((_BPW, _D), jnp.float32),
            pltpu.VMEM((2 * _D,), jnp.float32),
            pltpu.VMEM((128,), jnp.float32),
            pltpu.VMEM((_BPW,), jnp.float32),
            pltpu.SemaphoreType.DMA,
        ],
    )
    return f(user_idx.astype(jnp.int32), item_idx.astype(jnp.int32),
             user_table, item_table, wf, b128)


# speedup vs baseline: 3.3797x; 2.0365x over previous
import jax
import jax.numpy as jnp
from jax import lax
from jax.experimental import pallas as pl
from jax.experimental.pallas import tpu as pltpu
from jax.experimental.pallas import tpu_sc as plsc

_B = 16384
_D = 64
_NW = 32
_BPW = _B // _NW
_ICH = 128
_NCH = _BPW // _ICH
_GROUPS = _BPW // 16

_NUP = 1024000
_NIP = 102400
_UBW = 51200
_IBW = 51200


def _matvec_body(w_ref, tab_ref, out_ref):
    out_ref[...] = jnp.dot(w_ref[...], tab_ref[...],
                           precision=lax.Precision.HIGHEST)[0]


def _row_scores(tab_t, w_half, n_pad, bw):
    nb = n_pad // bw
    return pl.pallas_call(
        _matvec_body,
        grid=(nb,),
        in_specs=[
            pl.BlockSpec((1, _D), lambda i: (0, 0)),
            pl.BlockSpec((_D, bw), lambda i: (0, i)),
        ],
        out_specs=pl.BlockSpec((bw,), lambda i: (i,)),
        out_shape=jax.ShapeDtypeStruct((n_pad,), jnp.float32),
    )(w_half, tab_t)


def _sc_body(uidx_hbm, iidx_hbm, us_hbm, is_hbm, b_hbm, out_hbm,
             uidx_v, iidx_v, g_v, rows_v, b_v, out_v, sem):
    wid = lax.axis_index("s") * 2 + lax.axis_index("c")
    base = wid * _BPW

    pltpu.sync_copy(uidx_hbm.at[pl.ds(base, _BPW)], uidx_v)
    pltpu.sync_copy(iidx_hbm.at[pl.ds(base, _BPW)], iidx_v)
    pltpu.sync_copy(b_hbm, b_v)

    lane = lax.broadcasted_iota(jnp.int32, (16,), 0)
    bv = b_v[pl.ds(0, 16)]

    def phase(s_hbm, idx_v, first):
        def row_idx(c, carry):
            g_v[pl.ds(c * 16, 16)] = lax.shift_right_logical(
                idx_v[pl.ds(c * 16, 16)], 7)
            return carry
        lax.fori_loop(0, _GROUPS, row_idx, 0)

        copies = [
            pltpu.async_copy(s_hbm.at[g_v.at[pl.ds(k * _ICH, _ICH)]],
                             rows_v.at[pl.ds(k * _ICH, _ICH)], sem)
            for k in range(_NCH)
        ]
        for c in copies:
            c.wait()

        def group(g, carry):
            mvals = idx_v[pl.ds(g * 16, 16)] & 127
            acc = jnp.zeros((16,), jnp.float32)
            for rr in range(16):
                r = g * 16 + rr
                m = mvals.at[jnp.full((16,), rr, jnp.int32)].get(
                    mode="promise_in_bounds")
                p = jnp.zeros((16,), jnp.float32)
                for c in range(8):
                    oh = jnp.where(lane + c * 16 == m, 1.0, 0.0)
                    p = p + rows_v[r, pl.ds(c * 16, 16)] * oh
                for d in (1, 2, 4, 8):
                    p = p + p.at[lane ^ d].get(mode="promise_in_bounds")
                acc = jnp.where(lane == rr, p, acc)
            if first:
                out_v[pl.ds(g * 16, 16)] = acc + bv
            else:
                out_v[pl.ds(g * 16, 16)] = out_v[pl.ds(g * 16, 16)] + acc
            return carry
        lax.fori_loop(0, _GROUPS, group, 0)

    phase(us_hbm, uidx_v, True)
    phase(is_hbm, iidx_v, False)

    pltpu.sync_copy(out_v, out_hbm.at[pl.ds(base, _BPW)])


def kernel(user_idx, item_idx, user_table, item_table, W, b):
    wu = W[:, :_D].astype(jnp.float32)
    wi = W[:, _D:].astype(jnp.float32)
    b128 = jnp.broadcast_to(b.astype(jnp.float32), (128,))

    uscore = _row_scores(user_table.T, wu, _NUP, _UBW)
    iscore = _row_scores(item_table.T, wi, _NIP, _IBW)
    us2 = uscore.reshape(_NUP // 128, 128)
    is2 = iscore.reshape(_NIP // 128, 128)

    mesh = plsc.VectorSubcoreMesh(core_axis_name="c", subcore_axis_name="s")
    f = pl.kernel(
        _sc_body,
        out_type=jax.ShapeDtypeStruct((_B,), jnp.float32),
        mesh=mesh,
        compiler_params=pltpu.CompilerParams(use_tc_tiling_on_sc=True),
        scratch_types=[
            pltpu.VMEM((_BPW,), jnp.int32),
            pltpu.VMEM((_BPW,), jnp.int32),
            pltpu.VMEM((_BPW,), jnp.int32),
            pltpu.VMEM((_BPW, 128), jnp.float32),
            pltpu.VMEM((128,), jnp.float32),
            pltpu.VMEM((_BPW,), jnp.float32),
            pltpu.SemaphoreType.DMA,
        ],
    )
    return f(user_idx.astype(jnp.int32), item_idx.astype(jnp.int32),
             us2, is2, b128)


# speedup vs baseline: 4.0845x; 1.2085x over previous
import jax
import jax.numpy as jnp
from jax import lax
from jax.experimental import pallas as pl
from jax.experimental.pallas import tpu as pltpu
from jax.experimental.pallas import tpu_sc as plsc

_B = 16384
_D = 64
_NW = 32
_BPW = _B // _NW
_ICH = 128
_NCH = _BPW // _ICH
_GROUPS = _BPW // 16

_NUP = 1024000
_NIP = 102400
_UBW = 102400
_IBW = 51200


def _matvec_body(w_ref, tab_ref, out_ref):
    out_ref[...] = jnp.dot(w_ref[...], tab_ref[...],
                           precision=lax.Precision.HIGHEST)[0]


def _row_scores(tab_t, w_half, n_pad, bw):
    nb = n_pad // bw
    return pl.pallas_call(
        _matvec_body,
        grid=(nb,),
        in_specs=[
            pl.BlockSpec((1, _D), lambda i: (0, 0)),
            pl.BlockSpec((_D, bw), lambda i: (0, i)),
        ],
        out_specs=pl.BlockSpec((bw,), lambda i: (i,)),
        out_shape=jax.ShapeDtypeStruct((n_pad,), jnp.float32),
    )(w_half, tab_t)


def _sc_body(idx_hbm, s_hbm, add_hbm, out_hbm,
             idx_v, g_v, rows_v, add_v, out_v, sem):
    wid = lax.axis_index("s") * 2 + lax.axis_index("c")
    base = wid * _BPW

    pltpu.sync_copy(idx_hbm.at[pl.ds(base, _BPW)], idx_v)
    pltpu.sync_copy(add_hbm.at[pl.ds(base, _BPW)], add_v)

    lane = lax.broadcasted_iota(jnp.int32, (16,), 0)

    def row_idx(c, carry):
        g_v[pl.ds(c * 16, 16)] = lax.shift_right_logical(
            idx_v[pl.ds(c * 16, 16)], 7)
        return carry
    lax.fori_loop(0, _GROUPS, row_idx, 0)

    copies = [
        pltpu.async_copy(s_hbm.at[g_v.at[pl.ds(k * _ICH, _ICH)]],
                         rows_v.at[pl.ds(k * _ICH, _ICH)], sem)
        for k in range(_NCH)
    ]
    for c in copies:
        c.wait()

    def group(g, carry):
        mvals = idx_v[pl.ds(g * 16, 16)] & 127
        acc = jnp.zeros((16,), jnp.float32)
        for rr in range(16):
            r = g * 16 + rr
            m = mvals.at[jnp.full((16,), rr, jnp.int32)].get(
                mode="promise_in_bounds")
            mlow = m & 15
            m1 = (lax.shift_right_logical(m, 4) & 1).astype(jnp.float32)
            m2 = (lax.shift_right_logical(m, 5) & 1).astype(jnp.float32)
            m3 = lax.shift_right_logical(m, 6).astype(jnp.float32)
            rv = [rows_v[r, pl.ds(c * 16, 16)] for c in range(8)]
            q = [rv[2 * i] + m1 * (rv[2 * i + 1] - rv[2 * i])
                 for i in range(4)]
            s = [q[2 * i] + m2 * (q[2 * i + 1] - q[2 * i])
                 for i in range(2)]
            t = s[0] + m3 * (s[1] - s[0])
            word = t.at[mlow].get(mode="promise_in_bounds")
            acc = jnp.where(lane == rr, word, acc)
        out_v[pl.ds(g * 16, 16)] = acc + add_v[pl.ds(g * 16, 16)]
        return carry
    lax.fori_loop(0, _GROUPS, group, 0)

    pltpu.sync_copy(out_v, out_hbm.at[pl.ds(base, _BPW)])


def _sc_gather_add(idx, score2d, addend):
    mesh = plsc.VectorSubcoreMesh(core_axis_name="c", subcore_axis_name="s")
    f = pl.kernel(
        _sc_body,
        out_type=jax.ShapeDtypeStruct((_B,), jnp.float32),
        mesh=mesh,
        compiler_params=pltpu.CompilerParams(use_tc_tiling_on_sc=True),
        scratch_types=[
            pltpu.VMEM((_BPW,), jnp.int32),
            pltpu.VMEM((_BPW,), jnp.int32),
            pltpu.VMEM((_BPW, 128), jnp.float32),
            pltpu.VMEM((_BPW,), jnp.float32),
            pltpu.VMEM((_BPW,), jnp.float32),
            pltpu.SemaphoreType.DMA,
        ],
    )
    return f(idx, score2d, addend)


def kernel(user_idx, item_idx, user_table, item_table, W, b):
    wu = W[:, :_D].astype(jnp.float32)
    wi = W[:, _D:].astype(jnp.float32)
    bb = jnp.broadcast_to(b.astype(jnp.float32), (_B,))

    iscore = _row_scores(item_table.T, wi, _NIP, _IBW)
    is2 = iscore.reshape(_NIP // 128, 128)
    partial = _sc_gather_add(item_idx.astype(jnp.int32), is2, bb)

    uscore = _row_scores(user_table.T, wu, _NUP, _UBW)
    us2 = uscore.reshape(_NUP // 128, 128)
    return _sc_gather_add(user_idx.astype(jnp.int32), us2, partial)


# speedup vs baseline: 4.2341x; 1.0366x over previous
import jax
import jax.numpy as jnp
from jax import lax
from jax.experimental import pallas as pl
from jax.experimental.pallas import tpu as pltpu
from jax.experimental.pallas import tpu_sc as plsc

_B = 16384
_D = 64
_NW = 32
_BPW = _B // _NW
_ICH = 128
_NCH = _BPW // _ICH
_GROUPS = _BPW // 16

_NUP = 1024000
_NIP = 102400
_UBW = 51200
_IBW = 25600


def _matvec_body(w_ref, tab_ref, out_ref):
    out_ref[...] = jnp.dot(w_ref[...], tab_ref[...],
                           precision=lax.Precision.HIGHEST)[0]


def _row_scores(tab_t, w_half, n_pad, bw):
    nb = n_pad // bw
    return pl.pallas_call(
        _matvec_body,
        grid=(nb,),
        in_specs=[
            pl.BlockSpec((1, _D), lambda i: (0, 0)),
            pl.BlockSpec((_D, bw), lambda i: (0, i)),
        ],
        out_specs=pl.BlockSpec((bw,), lambda i: (i,)),
        out_shape=jax.ShapeDtypeStruct((n_pad,), jnp.float32),
    )(w_half, tab_t)


def _sc_body(idx_hbm, s_hbm, add_hbm, out_hbm,
             idx_v, g_v, rows_v, add_v, out_v, sem):
    wid = lax.axis_index("s") * 2 + lax.axis_index("c")
    base = wid * _BPW

    pltpu.sync_copy(idx_hbm.at[pl.ds(base, _BPW)], idx_v)
    pltpu.sync_copy(add_hbm.at[pl.ds(base, _BPW)], add_v)

    lane = lax.broadcasted_iota(jnp.int32, (16,), 0)

    def row_idx(c, carry):
        g_v[pl.ds(c * 16, 16)] = lax.shift_right_logical(
            idx_v[pl.ds(c * 16, 16)], 7)
        return carry
    lax.fori_loop(0, _GROUPS, row_idx, 0)

    copies = [
        pltpu.async_copy(s_hbm.at[g_v.at[pl.ds(k * _ICH, _ICH)]],
                         rows_v.at[pl.ds(k * _ICH, _ICH)], sem)
        for k in range(_NCH)
    ]
    for c in copies:
        c.wait()

    def group(g, carry):
        mvals = idx_v[pl.ds(g * 16, 16)] & 127
        acc = jnp.zeros((16,), jnp.float32)
        for rr in range(16):
            r = g * 16 + rr
            m = mvals.at[jnp.full((16,), rr, jnp.int32)].get(
                mode="promise_in_bounds")
            mlow = m & 15
            m1 = (lax.shift_right_logical(m, 4) & 1).astype(jnp.float32)
            m2 = (lax.shift_right_logical(m, 5) & 1).astype(jnp.float32)
            m3 = lax.shift_right_logical(m, 6).astype(jnp.float32)
            rv = [rows_v[r, pl.ds(c * 16, 16)] for c in range(8)]
            q = [rv[2 * i] + m1 * (rv[2 * i + 1] - rv[2 * i])
                 for i in range(4)]
            s = [q[2 * i] + m2 * (q[2 * i + 1] - q[2 * i])
                 for i in range(2)]
            t = s[0] + m3 * (s[1] - s[0])
            word = t.at[mlow].get(mode="promise_in_bounds")
            acc = jnp.where(lane == rr, word, acc)
        out_v[pl.ds(g * 16, 16)] = acc + add_v[pl.ds(g * 16, 16)]
        return carry
    lax.fori_loop(0, _GROUPS, group, 0)

    pltpu.sync_copy(out_v, out_hbm.at[pl.ds(base, _BPW)])


def _sc_gather_add(idx, score2d, addend):
    mesh = plsc.VectorSubcoreMesh(core_axis_name="c", subcore_axis_name="s")
    f = pl.kernel(
        _sc_body,
        out_type=jax.ShapeDtypeStruct((_B,), jnp.float32),
        mesh=mesh,
        compiler_params=pltpu.CompilerParams(use_tc_tiling_on_sc=True),
        scratch_types=[
            pltpu.VMEM((_BPW,), jnp.int32),
            pltpu.VMEM((_BPW,), jnp.int32),
            pltpu.VMEM((_BPW, 128), jnp.float32),
            pltpu.VMEM((_BPW,), jnp.float32),
            pltpu.VMEM((_BPW,), jnp.float32),
            pltpu.SemaphoreType.DMA,
        ],
    )
    return f(idx, score2d, addend)


def kernel(user_idx, item_idx, user_table, item_table, W, b):
    wu = W[:, :_D].astype(jnp.float32)
    wi = W[:, _D:].astype(jnp.float32)
    bb = jnp.broadcast_to(b.astype(jnp.float32), (_B,))

    iscore = _row_scores(item_table.T, wi, _NIP, _IBW)
    is2 = iscore.reshape(_NIP // 128, 128)
    partial = _sc_gather_add(item_idx.astype(jnp.int32), is2, bb)

    uscore = _row_scores(user_table.T, wu, _NUP, _UBW)
    us2 = uscore.reshape(_NUP // 128, 128)
    return _sc_gather_add(user_idx.astype(jnp.int32), us2, partial)
